# baseline (speedup 1.0000x reference)
import jax
import jax.numpy as jnp
from jax.experimental import pallas as pl
from jax.experimental.pallas import tpu as pltpu

_N = 4096
_IN = 512
_OUT = 512
_KT = _N + _IN
_BLK = 512
_M = _N // _BLK
_S = 2
_CH = _BLK // _S
_XCH = _N // 2
_EPS = 1e-5


def _fused_kernel(am_hbm, x1_hbm, x2_hbm, w_ref, gamma_ref, beta_ref,
                  out_hbm, abuf, x1buf, ybuf, t_ref, sum_ref, sumsq_ref,
                  asem, xsem, osem):
    p = pl.program_id(0)

    def am_copy(block, buf, s):
        return pltpu.make_async_copy(
            am_hbm.at[pl.ds(block * _BLK + s * _CH, _CH), :],
            abuf.at[buf, pl.ds(s * _CH, _CH), 0:_N],
            asem.at[buf, s])

    def x2_copy(block, buf):
        return pltpu.make_async_copy(
            x2_hbm.at[pl.ds(block * _BLK, _BLK), :],
            abuf.at[buf, :, pl.ds(_N, _IN)],
            asem.at[buf, _S])

    def x1_copy(s):
        return pltpu.make_async_copy(
            x1_hbm.at[pl.ds(s * _XCH, _XCH), :],
            x1buf.at[s, :, :],
            xsem.at[s])

    @pl.when(p == 0)
    def _stage():
        x1_copy(0).start()
        x1_copy(1).start()
        t_ref[pl.ds(_N, _IN), :] = w_ref[_IN:2 * _IN, :].astype(jnp.bfloat16)
        w1 = w_ref[0:_IN, :].astype(jnp.bfloat16)
        for s in range(2):
            x1_copy(s).wait()
            am_copy(0, 0, s).start()
            x1b = x1buf[s].astype(jnp.bfloat16)
            t = jnp.dot(x1b, w1, preferred_element_type=jnp.float32)
            t_ref[pl.ds(s * _XCH, _XCH), :] = t.astype(jnp.bfloat16)
        x2_copy(0, 0).start()
        for s in range(_S):
            am_copy(1, 1, s).start()
        x2_copy(1, 1).start()

    @pl.when(p > 0)
    def _compute():
        m = p - 1
        buf = jax.lax.rem(m, 2)
        for s in range(_S):
            am_copy(m, buf, s).wait()
        x2_copy(m, buf).wait()
        lhs = abuf[buf].astype(jnp.bfloat16)
        y = jnp.dot(lhs, t_ref[...], preferred_element_type=jnp.float32)
        ybuf[pl.ds(m * _BLK, _BLK), :] = y

        ps = jnp.sum(y, axis=0, keepdims=True)
        pss = jnp.sum(y * y, axis=0, keepdims=True)

        @pl.when(m == 0)
        def _init_stats():
            sum_ref[...] = ps
            sumsq_ref[...] = pss

        @pl.when(m > 0)
        def _acc_stats():
            sum_ref[...] += ps
            sumsq_ref[...] += pss

        @pl.when(p < _M - 1)
        def _issue_next():
            for s in range(_S):
                am_copy(p + 1, buf, s).start()
            x2_copy(p + 1, buf).start()

    @pl.when(p == _M)
    def _normalize():
        inv_n = 1.0 / _N
        mean = sum_ref[...] * inv_n
        var = sumsq_ref[...] * inv_n - mean * mean
        scale = jax.lax.rsqrt(var + _EPS) * gamma_ref[...]
        shift = beta_ref[...] - mean * scale
        for i in range(_M):
            blk = ybuf[pl.ds(i * _BLK, _BLK), :]
            ybuf[pl.ds(i * _BLK, _BLK), :] = jnp.maximum(
                blk * scale + shift, 0.0)
            pltpu.make_async_copy(
                ybuf.at[pl.ds(i * _BLK, _BLK), :],
                out_hbm.at[pl.ds(i * _BLK, _BLK), :],
                osem.at[i]).start()
        for i in range(_M):
            pltpu.make_async_copy(
                ybuf.at[pl.ds(i * _BLK, _BLK), :],
                out_hbm.at[pl.ds(i * _BLK, _BLK), :],
                osem.at[i]).wait()


def kernel(x1, x2, am, W, b, gamma, beta):
    del b
    g2 = jnp.reshape(gamma, (1, _OUT))
    be2 = jnp.reshape(beta, (1, _OUT))

    out = pl.pallas_call(
        _fused_kernel,
        grid=(_M + 1,),
        in_specs=[
            pl.BlockSpec(memory_space=pltpu.MemorySpace.HBM),
            pl.BlockSpec(memory_space=pltpu.MemorySpace.HBM),
            pl.BlockSpec(memory_space=pltpu.MemorySpace.HBM),
            pl.BlockSpec((2 * _IN, _OUT), lambda p: (0, 0)),
            pl.BlockSpec((1, _OUT), lambda p: (0, 0)),
            pl.BlockSpec((1, _OUT), lambda p: (0, 0)),
        ],
        out_specs=pl.BlockSpec(memory_space=pltpu.MemorySpace.HBM),
        out_shape=jax.ShapeDtypeStruct((_N, _OUT), jnp.float32),
        scratch_shapes=[
            pltpu.VMEM((2, _BLK, _KT), jnp.float32),
            pltpu.VMEM((2, _XCH, _IN), jnp.float32),
            pltpu.VMEM((_N, _OUT), jnp.float32),
            pltpu.VMEM((_KT, _OUT), jnp.bfloat16),
            pltpu.VMEM((1, _OUT), jnp.float32),
            pltpu.VMEM((1, _OUT), jnp.float32),
            pltpu.SemaphoreType.DMA((2, _S + 1)),
            pltpu.SemaphoreType.DMA((2,)),
            pltpu.SemaphoreType.DMA((_M,)),
        ],
        compiler_params=pltpu.CompilerParams(
            dimension_semantics=("arbitrary",)),
    )(am, x1, x2, W, g2, be2)
    return out


# speedup vs baseline: 1.0102x; 1.0102x over previous
import jax
import jax.numpy as jnp
from jax.experimental import pallas as pl
from jax.experimental.pallas import tpu as pltpu

_N = 4096
_IN = 512
_OUT = 512
_KT = _N + _IN
_BLK = 512
_M = _N // _BLK
_S = 2
_CH = _BLK // _S
_XCH = _N // 2
_EPS = 1e-5


def _fused_kernel(am_hbm, x1_hbm, x2_hbm, w_ref, gamma_ref, beta_ref,
                  out_hbm, abuf, x1buf, ybuf, t_ref, sum_ref, sumsq_ref,
                  asem, xsem, osem):
    p = pl.program_id(0)

    def am_copy(block, buf, s):
        return pltpu.make_async_copy(
            am_hbm.at[pl.ds(block * _BLK + s * _CH, _CH), :],
            abuf.at[buf, pl.ds(s * _CH, _CH), 0:_N],
            asem.at[buf, s])

    def x2_copy(block, buf):
        return pltpu.make_async_copy(
            x2_hbm.at[pl.ds(block * _BLK, _BLK), :],
            abuf.at[buf, :, pl.ds(_N, _IN)],
            asem.at[buf, _S])

    def am0_col_copy(half):
        return pltpu.make_async_copy(
            am_hbm.at[0:_BLK, pl.ds(half * (_N // 2), _N // 2)],
            abuf.at[0, :, pl.ds(half * (_N // 2), _N // 2)],
            asem.at[0, half])

    def x1_copy(s):
        return pltpu.make_async_copy(
            x1_hbm.at[pl.ds(s * _XCH, _XCH), :],
            x1buf.at[s, :, :],
            xsem.at[s])

    @pl.when(p == 0)
    def _stage():
        x1_copy(0).start()
        x1_copy(1).start()
        t_ref[pl.ds(_N, _IN), :] = w_ref[_IN:2 * _IN, :].astype(jnp.bfloat16)
        w1 = w_ref[0:_IN, :].astype(jnp.bfloat16)
        for s in range(2):
            x1_copy(s).wait()
            am0_col_copy(s).start()
            x1b = x1buf[s].astype(jnp.bfloat16)
            t = jnp.dot(x1b, w1, preferred_element_type=jnp.float32)
            t_ref[pl.ds(s * _XCH, _XCH), :] = t.astype(jnp.bfloat16)
        x2_copy(0, 0).start()
        for s in range(_S):
            am_copy(1, 1, s).start()
        x2_copy(1, 1).start()

    @pl.when(p == 1)
    def _compute_first():
        am0_col_copy(0).wait()
        lhs_a = abuf[0, :, 0:(_N // 2)].astype(jnp.bfloat16)
        y_a = jnp.dot(lhs_a, t_ref[0:(_N // 2), :],
                      preferred_element_type=jnp.float32)
        am0_col_copy(1).wait()
        x2_copy(0, 0).wait()
        lhs_b = abuf[0, :, pl.ds(_N // 2, _KT - _N // 2)].astype(jnp.bfloat16)
        y_b = jnp.dot(lhs_b, t_ref[pl.ds(_N // 2, _KT - _N // 2), :],
                      preferred_element_type=jnp.float32)
        y = y_a + y_b
        ybuf[0:_BLK, :] = y
        sum_ref[...] = jnp.sum(y, axis=0, keepdims=True)
        sumsq_ref[...] = jnp.sum(y * y, axis=0, keepdims=True)
        for s in range(_S):
            am_copy(2, 0, s).start()
        x2_copy(2, 0).start()

    @pl.when(p > 1)
    def _compute():
        m = p - 1
        buf = jax.lax.rem(m, 2)
        for s in range(_S):
            am_copy(m, buf, s).wait()
        x2_copy(m, buf).wait()
        lhs = abuf[buf].astype(jnp.bfloat16)
        y = jnp.dot(lhs, t_ref[...], preferred_element_type=jnp.float32)
        ybuf[pl.ds(m * _BLK, _BLK), :] = y

        sum_ref[...] += jnp.sum(y, axis=0, keepdims=True)
        sumsq_ref[...] += jnp.sum(y * y, axis=0, keepdims=True)

        @pl.when(p < _M - 1)
        def _issue_next():
            for s in range(_S):
                am_copy(p + 1, buf, s).start()
            x2_copy(p + 1, buf).start()

    @pl.when(p == _M)
    def _normalize():
        inv_n = 1.0 / _N
        mean = sum_ref[...] * inv_n
        var = sumsq_ref[...] * inv_n - mean * mean
        scale = jax.lax.rsqrt(var + _EPS) * gamma_ref[...]
        shift = beta_ref[...] - mean * scale
        for i in range(_M):
            blk = ybuf[pl.ds(i * _BLK, _BLK), :]
            ybuf[pl.ds(i * _BLK, _BLK), :] = jnp.maximum(
                blk * scale + shift, 0.0)
            pltpu.make_async_copy(
                ybuf.at[pl.ds(i * _BLK, _BLK), :],
                out_hbm.at[pl.ds(i * _BLK, _BLK), :],
                osem.at[i]).start()
        for i in range(_M):
            pltpu.make_async_copy(
                ybuf.at[pl.ds(i * _BLK, _BLK), :],
                out_hbm.at[pl.ds(i * _BLK, _BLK), :],
                osem.at[i]).wait()


def kernel(x1, x2, am, W, b, gamma, beta):
    del b
    g2 = jnp.reshape(gamma, (1, _OUT))
    be2 = jnp.reshape(beta, (1, _OUT))

    out = pl.pallas_call(
        _fused_kernel,
        grid=(_M + 1,),
        in_specs=[
            pl.BlockSpec(memory_space=pltpu.MemorySpace.HBM),
            pl.BlockSpec(memory_space=pltpu.MemorySpace.HBM),
            pl.BlockSpec(memory_space=pltpu.MemorySpace.HBM),
            pl.BlockSpec((2 * _IN, _OUT), lambda p: (0, 0)),
            pl.BlockSpec((1, _OUT), lambda p: (0, 0)),
            pl.BlockSpec((1, _OUT), lambda p: (0, 0)),
        ],
        out_specs=pl.BlockSpec(memory_space=pltpu.MemorySpace.HBM),
        out_shape=jax.ShapeDtypeStruct((_N, _OUT), jnp.float32),
        scratch_shapes=[
            pltpu.VMEM((2, _BLK, _KT), jnp.float32),
            pltpu.VMEM((2, _XCH, _IN), jnp.float32),
            pltpu.VMEM((_N, _OUT), jnp.float32),
            pltpu.VMEM((_KT, _OUT), jnp.bfloat16),
            pltpu.VMEM((1, _OUT), jnp.float32),
            pltpu.VMEM((1, _OUT), jnp.float32),
            pltpu.SemaphoreType.DMA((2, _S + 1)),
            pltpu.SemaphoreType.DMA((2,)),
            pltpu.SemaphoreType.DMA((_M,)),
        ],
        compiler_params=pltpu.CompilerParams(
            dimension_semantics=("arbitrary",)),
    )(am, x1, x2, W, g2, be2)
    return out
